# Initial kernel scaffold; baseline (speedup 1.0000x reference)
import jax
import jax.numpy as jnp
from jax.experimental import pallas as pl


def kernel(x, edge_index, batch, params):
    raise NotImplementedError("write your pallas kernel here")


# speedup vs baseline: 24.5330x; 24.5330x over previous
import functools

import jax
import jax.numpy as jnp
from jax import lax
from jax.experimental import pallas as pl
from jax.experimental.pallas import tpu as pltpu
from jax.experimental.pallas import tpu_sc as plsc

NC, NS, LANES = 2, 16, 16
NW = NC * NS

_SC_PARAMS = pltpu.CompilerParams(needs_layout_passes=False)
_MESH = plsc.VectorSubcoreMesh(core_axis_name="c", subcore_axis_name="s")


def _fill(ref, n, value, dtype):
    v = jnp.full((LANES,), value, dtype)

    def body(i, _):
        ref[pl.ds(i * LANES, LANES)] = v
        return 0

    lax.fori_loop(0, n // LANES, body, 0)


def _merge_into(dst, src, n, op):
    def body(i, _):
        s = pl.ds(i * LANES, LANES)
        dst[s] = op(dst[s], src[s])
        return 0

    lax.fori_loop(0, n // LANES, body, 0)


def _scatter_max(ref, idx16, val16, mask=None):
    pend = val16 > plsc.load_gather(ref, [idx16])
    if mask is not None:
        pend = pend & mask

    def cond(p):
        return jnp.any(p)

    def step(p):
        plsc.store_scatter(ref, [idx16], val16, mask=p)
        newp = val16 > plsc.load_gather(ref, [idx16])
        if mask is not None:
            newp = newp & mask
        return newp

    lax.while_loop(cond, step, pend)


def _reduce_tiles(local, sh, out_hbm, n_pad, cid, sid, red, tmp, rps, op):
    pltpu.sync_copy(local, sh.at[sid])
    plsc.subcore_barrier()
    sl = pl.ds(sid * rps, rps)
    pltpu.sync_copy(sh.at[0, sl], red)
    for t in range(1, NS):
        pltpu.sync_copy(sh.at[t, sl], tmp)
        _merge_into(red, tmp, rps, op)
    pltpu.sync_copy(red, out_hbm.at[pl.ds(cid * n_pad + sid * rps, rps)])
    plsc.subcore_barrier()


def _edge_scatter(src, row, col, n_pad):
    n, d = src.shape
    e = row.shape[0]
    per_w = e // NW
    nfull = per_w // 128
    tail = per_w % 128
    rps = n_pad // NS

    @functools.partial(
        pl.kernel,
        out_type=jax.ShapeDtypeStruct((NC, n_pad, d), jnp.float32),
        mesh=_MESH,
        scratch_types=[
            pltpu.VMEM((128,), jnp.int32),
            pltpu.VMEM((128,), jnp.int32),
            pltpu.VMEM((128, d), jnp.float32),
            pltpu.VMEM((max(tail, 8),), jnp.int32),
            pltpu.VMEM((max(tail, 8),), jnp.int32),
            pltpu.VMEM_SHARED((n_pad, d), jnp.float32),
            pltpu.SemaphoreType.DMA,
        ],
        compiler_params=_SC_PARAMS,
    )
    def k(src_hbm, row_hbm, col_hbm, out_hbm, ridx, cidx, buf, rt, ct, acc, sem):
        cid = lax.axis_index("c")
        sid = lax.axis_index("s")
        wid = sid * NC + cid

        def zrow(i, _):
            for j in range(d // LANES):
                buf[i, pl.ds(j * LANES, LANES)] = jnp.zeros((LANES,), jnp.float32)
            return 0

        lax.fori_loop(0, 128, zrow, 0)
        for kk in range(rps // 128):
            pltpu.sync_copy(buf, acc.at[pl.ds(sid * rps + kk * 128, 128)])
        plsc.subcore_barrier()

        wbase = wid * per_w

        def blk(j, _):
            base = wbase + j * 128
            pltpu.sync_copy(row_hbm.at[pl.ds(base, 128)], ridx)
            pltpu.sync_copy(col_hbm.at[pl.ds(base, 128)], cidx)
            pltpu.async_copy(src_hbm.at[ridx], buf, sem).wait()
            pltpu.sync_copy(buf, acc.at[cidx], add=True)
            pltpu.async_copy(src_hbm.at[cidx], buf, sem).wait()
            pltpu.sync_copy(buf, acc.at[ridx], add=True)
            return 0

        lax.fori_loop(0, nfull, blk, 0)
        if tail:
            base = wbase + nfull * 128
            pltpu.sync_copy(row_hbm.at[pl.ds(base, tail)], rt)
            pltpu.sync_copy(col_hbm.at[pl.ds(base, tail)], ct)
            tbuf = buf.at[pl.ds(0, tail)]
            pltpu.async_copy(src_hbm.at[rt], tbuf, sem).wait()
            pltpu.sync_copy(tbuf, acc.at[ct], add=True)
            pltpu.async_copy(src_hbm.at[ct], tbuf, sem).wait()
            pltpu.sync_copy(tbuf, acc.at[rt], add=True)
        plsc.subcore_barrier()
        sl = pl.ds(sid * rps, rps)
        pltpu.sync_copy(acc.at[sl], out_hbm.at[cid, sl])

    return k(src, row, col)


def _row_scatter(src_pad, cluster, n_pad):
    n, d = src_pad.shape
    nblk = n // 128
    rps = n_pad // NS

    @functools.partial(
        pl.kernel,
        out_type=jax.ShapeDtypeStruct((NC, n_pad, d), jnp.float32),
        mesh=_MESH,
        scratch_types=[
            pltpu.VMEM((128,), jnp.int32),
            pltpu.VMEM((128, d), jnp.float32),
            pltpu.VMEM_SHARED((n_pad, d), jnp.float32),
        ],
        compiler_params=_SC_PARAMS,
    )
    def k(src_hbm, cl_hbm, out_hbm, cidx, buf, acc):
        cid = lax.axis_index("c")
        sid = lax.axis_index("s")
        wid = sid * NC + cid

        def zrow(i, _):
            for j in range(d // LANES):
                buf[i, pl.ds(j * LANES, LANES)] = jnp.zeros((LANES,), jnp.float32)
            return 0

        lax.fori_loop(0, 128, zrow, 0)
        for kk in range(rps // 128):
            pltpu.sync_copy(buf, acc.at[pl.ds(sid * rps + kk * 128, 128)])
        plsc.subcore_barrier()

        nb = (nblk - wid + NW - 1) // NW

        def blk(j, _):
            b = wid + j * NW
            pltpu.sync_copy(cl_hbm.at[pl.ds(b * 128, 128)], cidx)
            pltpu.sync_copy(src_hbm.at[pl.ds(b * 128, 128)], buf)
            pltpu.sync_copy(buf, acc.at[cidx], add=True)
            return 0

        lax.fori_loop(0, nb, blk, 0)
        plsc.subcore_barrier()
        sl = pl.ds(sid * rps, rps)
        pltpu.sync_copy(acc.at[sl], out_hbm.at[cid, sl])

    return k(src_pad, cluster)


def _cluster_pass1(row, col, ns_pad, n_pad):
    e = row.shape[0]
    per_w = e // NW
    nfull = per_w // 128
    tail = per_w % 128
    rps = n_pad // NS

    @functools.partial(
        pl.kernel,
        out_type=(
            jax.ShapeDtypeStruct((NC * n_pad,), jnp.float32),
            jax.ShapeDtypeStruct((NC * n_pad,), jnp.float32),
        ),
        mesh=_MESH,
        scratch_types=[
            pltpu.VMEM((n_pad,), jnp.float32),
            pltpu.VMEM((n_pad,), jnp.float32),
            pltpu.VMEM((n_pad,), jnp.float32),
            pltpu.VMEM((128,), jnp.int32),
            pltpu.VMEM((128,), jnp.int32),
            pltpu.VMEM((max(tail, 8),), jnp.int32),
            pltpu.VMEM((max(tail, 8),), jnp.int32),
            pltpu.VMEM((rps,), jnp.float32),
            pltpu.VMEM((rps,), jnp.float32),
            pltpu.VMEM_SHARED((NS, n_pad), jnp.float32),
        ],
        compiler_params=_SC_PARAMS,
    )
    def k(row_hbm, col_hbm, ns_hbm, best_out, deg_out, ns_l, best_l, deg_l,
          ridx, cidx, rt, ct, red, tmp, sh):
        cid = lax.axis_index("c")
        sid = lax.axis_index("s")
        wid = sid * NC + cid
        ones16 = jnp.ones((LANES,), jnp.float32)

        pltpu.sync_copy(ns_hbm, ns_l)
        _fill(best_l, n_pad, -1.0, jnp.float32)
        _fill(deg_l, n_pad, 0.0, jnp.float32)

        def groups(rref, cref, ng):
            for g in range(ng):
                r16 = rref[pl.ds(g * LANES, LANES)]
                c16 = cref[pl.ds(g * LANES, LANES)]
                es = plsc.load_gather(ns_l, [r16]) * plsc.load_gather(ns_l, [c16])
                _scatter_max(best_l, r16, es)
                _scatter_max(best_l, c16, es)
                plsc.addupdate_scatter(deg_l, [r16], ones16)
                plsc.addupdate_scatter(deg_l, [c16], ones16)

        wbase = wid * per_w

        def blk(j, _):
            base = wbase + j * 128
            pltpu.sync_copy(row_hbm.at[pl.ds(base, 128)], ridx)
            pltpu.sync_copy(col_hbm.at[pl.ds(base, 128)], cidx)
            groups(ridx, cidx, 128 // LANES)
            return 0

        lax.fori_loop(0, nfull, blk, 0)
        if tail:
            base = wbase + nfull * 128
            pltpu.sync_copy(row_hbm.at[pl.ds(base, tail)], rt)
            pltpu.sync_copy(col_hbm.at[pl.ds(base, tail)], ct)
            groups(rt, ct, tail // LANES)

        _reduce_tiles(best_l, sh, best_out, n_pad, cid, sid, red, tmp, rps,
                      jnp.maximum)
        _reduce_tiles(deg_l, sh, deg_out, n_pad, cid, sid, red, tmp, rps,
                      jnp.add)

    return k(row, col, ns_pad)


def _cluster_pass2(row, col, ns_pad, best_part, n_pad):
    e = row.shape[0]
    per_w = e // NW
    nfull = per_w // 128
    tail = per_w % 128
    rps = n_pad // NS

    @functools.partial(
        pl.kernel,
        out_type=jax.ShapeDtypeStruct((NC * n_pad,), jnp.int32),
        mesh=_MESH,
        scratch_types=[
            pltpu.VMEM((n_pad,), jnp.float32),
            pltpu.VMEM((n_pad,), jnp.float32),
            pltpu.VMEM((n_pad,), jnp.float32),
            pltpu.VMEM((n_pad,), jnp.int32),
            pltpu.VMEM((128,), jnp.int32),
            pltpu.VMEM((128,), jnp.int32),
            pltpu.VMEM((max(tail, 8),), jnp.int32),
            pltpu.VMEM((max(tail, 8),), jnp.int32),
            pltpu.VMEM((rps,), jnp.int32),
            pltpu.VMEM((rps,), jnp.int32),
            pltpu.VMEM_SHARED((NS, n_pad), jnp.int32),
        ],
        compiler_params=_SC_PARAMS,
    )
    def k(row_hbm, col_hbm, ns_hbm, best_hbm, par_out, ns_l, best_l, btmp,
          parent_l, ridx, cidx, rt, ct, red, tmp, sh):
        cid = lax.axis_index("c")
        sid = lax.axis_index("s")
        wid = sid * NC + cid

        pltpu.sync_copy(ns_hbm, ns_l)
        pltpu.sync_copy(best_hbm.at[pl.ds(0, n_pad)], best_l)
        pltpu.sync_copy(best_hbm.at[pl.ds(n_pad, n_pad)], btmp)
        _merge_into(best_l, btmp, n_pad, jnp.maximum)
        _fill(parent_l, n_pad, -1, jnp.int32)

        def groups(rref, cref, ng):
            for g in range(ng):
                r16 = rref[pl.ds(g * LANES, LANES)]
                c16 = cref[pl.ds(g * LANES, LANES)]
                es = plsc.load_gather(ns_l, [r16]) * plsc.load_gather(ns_l, [c16])
                isb_r = es >= plsc.load_gather(best_l, [r16])
                _scatter_max(parent_l, r16, c16, mask=isb_r)
                isb_c = es >= plsc.load_gather(best_l, [c16])
                _scatter_max(parent_l, c16, r16, mask=isb_c)

        wbase = wid * per_w

        def blk(j, _):
            base = wbase + j * 128
            pltpu.sync_copy(row_hbm.at[pl.ds(base, 128)], ridx)
            pltpu.sync_copy(col_hbm.at[pl.ds(base, 128)], cidx)
            groups(ridx, cidx, 128 // LANES)
            return 0

        lax.fori_loop(0, nfull, blk, 0)
        if tail:
            base = wbase + nfull * 128
            pltpu.sync_copy(row_hbm.at[pl.ds(base, tail)], rt)
            pltpu.sync_copy(col_hbm.at[pl.ds(base, tail)], ct)
            groups(rt, ct, tail // LANES)

        _reduce_tiles(parent_l, sh, par_out, n_pad, cid, sid, red, tmp, rps,
                      jnp.maximum)

    return k(row, col, ns_pad, best_part)


def _cluster_pass3(parent_part, deg_part, ns_pad, n_pad):
    rps = n_pad // NS
    npw = n_pad // NW

    @functools.partial(
        pl.kernel,
        out_type=(
            jax.ShapeDtypeStruct((n_pad,), jnp.int32),
            jax.ShapeDtypeStruct((NC * n_pad,), jnp.float32),
            jax.ShapeDtypeStruct((NC * n_pad,), jnp.float32),
            jax.ShapeDtypeStruct((NC * n_pad,), jnp.float32),
        ),
        mesh=_MESH,
        scratch_types=[
            pltpu.VMEM((n_pad,), jnp.int32),
            pltpu.VMEM((n_pad,), jnp.int32),
            pltpu.VMEM((n_pad,), jnp.int32),
            pltpu.VMEM((n_pad,), jnp.float32),
            pltpu.VMEM((n_pad,), jnp.float32),
            pltpu.VMEM((n_pad,), jnp.float32),
            pltpu.VMEM((npw,), jnp.int32),
            pltpu.VMEM((npw,), jnp.float32),
            pltpu.VMEM((npw,), jnp.float32),
            pltpu.VMEM((npw,), jnp.float32),
            pltpu.VMEM((rps,), jnp.float32),
            pltpu.VMEM((rps,), jnp.float32),
            pltpu.VMEM_SHARED((NS, n_pad), jnp.float32),
        ],
        compiler_params=_SC_PARAMS,
    )
    def k(par_hbm, deg_hbm, ns_hbm, cl_out, cs_out, lk_out, sp_out,
          parent_l, itmp, cluster1_l, csize_l, link_l, spool_l,
          clbuf, dbuf, dtmp, nbuf, red, tmp, sh):
        cid = lax.axis_index("c")
        sid = lax.axis_index("s")
        wid = sid * NC + cid
        ones16 = jnp.ones((LANES,), jnp.float32)

        pltpu.sync_copy(par_hbm.at[pl.ds(0, n_pad)], parent_l)
        pltpu.sync_copy(par_hbm.at[pl.ds(n_pad, n_pad)], itmp)
        _merge_into(parent_l, itmp, n_pad, jnp.maximum)

        iota16 = lax.iota(jnp.int32, LANES)

        def c1(i, _):
            s = pl.ds(i * LANES, LANES)
            idxv = iota16 + i * LANES
            p = parent_l[s]
            p = jnp.where(p < 0, idxv, p)
            cluster1_l[s] = jnp.minimum(idxv, p)
            return 0

        lax.fori_loop(0, n_pad // LANES, c1, 0)

        _fill(csize_l, n_pad, 0.0, jnp.float32)
        _fill(link_l, n_pad, 0.0, jnp.float32)
        _fill(spool_l, n_pad, -1.0, jnp.float32)

        base = wid * npw
        pltpu.sync_copy(deg_hbm.at[pl.ds(base, npw)], dbuf)
        pltpu.sync_copy(deg_hbm.at[pl.ds(n_pad + base, npw)], dtmp)
        _merge_into(dbuf, dtmp, npw, jnp.add)
        pltpu.sync_copy(ns_hbm.at[pl.ds(base, npw)], nbuf)

        for g in range(npw // LANES):
            s = pl.ds(g * LANES, LANES)
            cl1v = cluster1_l[pl.ds(base + g * LANES, LANES)]
            clp = plsc.load_gather(cluster1_l, [cl1v])
            cl = jnp.minimum(cl1v, clp)
            clbuf[s] = cl
            plsc.addupdate_scatter(csize_l, [cl], ones16)
            plsc.addupdate_scatter(link_l, [cl], dbuf[s])
            _scatter_max(spool_l, cl, nbuf[s])

        pltpu.sync_copy(clbuf, cl_out.at[pl.ds(base, npw)])

        _reduce_tiles(csize_l, sh, cs_out, n_pad, cid, sid, red, tmp, rps,
                      jnp.add)
        _reduce_tiles(link_l, sh, lk_out, n_pad, cid, sid, red, tmp, rps,
                      jnp.add)
        _reduce_tiles(spool_l, sh, sp_out, n_pad, cid, sid, red, tmp, rps,
                      jnp.maximum)

    return k(parent_part, deg_part, ns_pad)


def _mlp2(h, w0, b0, w1, b1):
    h = jnp.maximum(jnp.dot(h, w0, preferred_element_type=jnp.float32) + b0, 0.0)
    return jnp.dot(h, w1, preferred_element_type=jnp.float32) + b1


def _full_spec(shape):
    nd = len(shape)
    return pl.BlockSpec(shape, lambda i: (0,) * nd)


def _tc_in_mlp(x, w0, b0, w1, b1, br):
    n, d = x.shape

    def body(x_r, w0_r, b0_r, w1_r, b1_r, o_r):
        o_r[...] = _mlp2(x_r[...], w0_r[...], b0_r[...], w1_r[...], b1_r[...])

    return pl.pallas_call(
        body,
        grid=(n // br,),
        in_specs=[
            pl.BlockSpec((br, d), lambda i: (i, 0)),
            _full_spec(w0.shape), _full_spec(b0.shape),
            _full_spec(w1.shape), _full_spec(b1.shape),
        ],
        out_specs=pl.BlockSpec((br, d), lambda i: (i, 0)),
        out_shape=jax.ShapeDtypeStruct((n, d), jnp.float32),
    )(x, w0, b0, w1, b1)


def _tc_gnn_layer(g, agg_part, w, b, br):
    n, d = g.shape

    def body(g_r, a0_r, a1_r, w_r, b_r, o_r):
        s = g_r[...] + a0_r[0] + a1_r[0]
        o_r[...] = jnp.maximum(
            jnp.dot(s, w_r[...], preferred_element_type=jnp.float32) + b_r[...],
            0.0)

    return pl.pallas_call(
        body,
        grid=(n // br,),
        in_specs=[
            pl.BlockSpec((br, d), lambda i: (i, 0)),
            pl.BlockSpec((1, br, d), lambda i: (0, i, 0)),
            pl.BlockSpec((1, br, d), lambda i: (1, i, 0)),
            _full_spec(w.shape), _full_spec(b.shape),
        ],
        out_specs=pl.BlockSpec((br, d), lambda i: (i, 0)),
        out_shape=jax.ShapeDtypeStruct((n, d), jnp.float32),
    )(g, agg_part, agg_part, w, b)


def _tc_gnn2_score_pre(g, agg_part, w, b, sw, sb, pw0, pb0, pw1, pb1, br):
    n, d = g.shape

    def body(g_r, a0_r, a1_r, w_r, b_r, sw_r, sb_r, pw0_r, pb0_r, pw1_r,
             pb1_r, ns_r, pre_r):
        s = g_r[...] + a0_r[0] + a1_r[0]
        g2 = jnp.maximum(
            jnp.dot(s, w_r[...], preferred_element_type=jnp.float32) + b_r[...],
            0.0)
        z = jnp.dot(g2, sw_r[...], preferred_element_type=jnp.float32) + sb_r[...]
        ns_r[...] = jax.nn.sigmoid(z)
        pre_r[...] = _mlp2(g2, pw0_r[...], pb0_r[...], pw1_r[...], pb1_r[...])

    return pl.pallas_call(
        body,
        grid=(n // br,),
        in_specs=[
            pl.BlockSpec((br, d), lambda i: (i, 0)),
            pl.BlockSpec((1, br, d), lambda i: (0, i, 0)),
            pl.BlockSpec((1, br, d), lambda i: (1, i, 0)),
            _full_spec(w.shape), _full_spec(b.shape),
            _full_spec(sw.shape), _full_spec(sb.shape),
            _full_spec(pw0.shape), _full_spec(pb0.shape),
            _full_spec(pw1.shape), _full_spec(pb1.shape),
        ],
        out_specs=(
            pl.BlockSpec((br, 1), lambda i: (i, 0)),
            pl.BlockSpec((br, d), lambda i: (i, 0)),
        ),
        out_shape=(
            jax.ShapeDtypeStruct((n, 1), jnp.float32),
            jax.ShapeDtypeStruct((n, d), jnp.float32),
        ),
    )(g, agg_part, agg_part, w, b, sw, sb, pw0, pb0, pw1, pb1)


def _tc_final(pooled_part, pw0, pb0, pw1, pb1, sp, lk, cs, h, ow0, ob0, ow1,
              ob1, br):
    n, d = h.shape

    def body(q0_r, q1_r, pw0_r, pb0_r, pw1_r, pb1_r, sp0_r, sp1_r, lk0_r,
             lk1_r, cs0_r, cs1_r, h_r, ow0_r, ob0_r, ow1_r, ob1_r, o_r):
        pooled = _mlp2(q0_r[0] + q1_r[0], pw0_r[...], pb0_r[...], pw1_r[...],
                       pb1_r[...])
        csv = cs0_r[0] + cs1_r[0]
        spool = jnp.maximum(sp0_r[0], sp1_r[0])
        spool = jnp.where(csv > 0.0, spool, 0.0)
        lkv = lk0_r[0] + lk1_r[0]
        pooled = pooled * spool * jnp.log1p(lkv)
        ho = jnp.where(csv == 1.0, h_r[...], pooled)
        o_r[...] = _mlp2(ho, ow0_r[...], ob0_r[...], ow1_r[...], ob1_r[...])

    def part2d(i_sel):
        return pl.BlockSpec((1, br, d), lambda i, s=i_sel: (s, i, 0))

    def part1d(i_sel):
        return pl.BlockSpec((1, br, 1), lambda i, s=i_sel: (s, i, 0))

    return pl.pallas_call(
        body,
        grid=(n // br,),
        in_specs=[
            part2d(0), part2d(1),
            _full_spec(pw0.shape), _full_spec(pb0.shape),
            _full_spec(pw1.shape), _full_spec(pb1.shape),
            part1d(0), part1d(1), part1d(0), part1d(1), part1d(0), part1d(1),
            pl.BlockSpec((br, d), lambda i: (i, 0)),
            _full_spec(ow0.shape), _full_spec(ob0.shape),
            _full_spec(ow1.shape), _full_spec(ob1.shape),
        ],
        out_specs=pl.BlockSpec((br, d), lambda i: (i, 0)),
        out_shape=jax.ShapeDtypeStruct((n, d), jnp.float32),
    )(pooled_part, pooled_part, pw0, pb0, pw1, pb1, sp, sp, lk, lk, cs, cs,
      h, ow0, ob0, ow1, ob1)


def kernel(x, edge_index, batch, params):
    n, d = x.shape
    n_pad = -(-n // 2048) * 2048
    row = edge_index[0]
    col = edge_index[1]
    p = params
    br = 2000 if n % 2000 == 0 else 400

    def b2d(b):
        return b.reshape(1, d)

    in_w, in_b = p["in_W"], p["in_b"]
    gnn_w, gnn_b = p["gnn_W"], p["gnn_b"]
    pre_w, pre_b = p["pre_W"], p["pre_b"]
    post_w, post_b = p["post_W"], p["post_b"]
    out_w, out_b = p["out_W"], p["out_b"]
    sw = p["score_w"].reshape(d, 1)
    sb = p["score_b"].reshape(1, 1)

    h = _tc_in_mlp(x, in_w[0], b2d(in_b[0]), in_w[1], b2d(in_b[1]), br)
    agg1 = _edge_scatter(h, row, col, n_pad)
    g1 = _tc_gnn_layer(h, agg1, gnn_w[0], b2d(gnn_b[0]), br)
    agg2 = _edge_scatter(g1, row, col, n_pad)
    ns, pre = _tc_gnn2_score_pre(
        g1, agg2, gnn_w[1], b2d(gnn_b[1]), sw, sb,
        pre_w[0], b2d(pre_b[0]), pre_w[1], b2d(pre_b[1]), br)

    ns_pad = jnp.pad(ns.reshape(-1), (0, n_pad - n))
    best_part, deg_part = _cluster_pass1(row, col, ns_pad, n_pad)
    parent_part = _cluster_pass2(row, col, ns_pad, best_part, n_pad)
    cluster_pad, cs_part, lk_part, sp_part = _cluster_pass3(
        parent_part, deg_part, ns_pad, n_pad)

    pre_pad = jnp.pad(pre, ((0, n_pad - n), (0, 0)))
    pooled_part = _row_scatter(pre_pad, cluster_pad, n_pad)

    out = _tc_final(
        pooled_part, post_w[0], b2d(post_b[0]), post_w[1], b2d(post_b[1]),
        sp_part.reshape(NC, n_pad, 1), lk_part.reshape(NC, n_pad, 1),
        cs_part.reshape(NC, n_pad, 1), h,
        out_w[0], b2d(out_b[0]), out_w[1], b2d(out_b[1]), br)
    return out, cluster_pad[:n]


# speedup vs baseline: 33.4573x; 1.3638x over previous
import functools

import jax
import jax.numpy as jnp
from jax import lax
from jax.experimental import pallas as pl
from jax.experimental.pallas import tpu as pltpu
from jax.experimental.pallas import tpu_sc as plsc

NC, NS, LANES = 2, 16, 16
NW = NC * NS

_SC_PARAMS = pltpu.CompilerParams(needs_layout_passes=False)
_MESH = plsc.VectorSubcoreMesh(core_axis_name="c", subcore_axis_name="s")


def _fill(ref, n, value, dtype):
    v = jnp.full((LANES,), value, dtype)

    def body(i, _):
        ref[pl.ds(i * LANES, LANES)] = v
        return 0

    lax.fori_loop(0, n // LANES, body, 0)


def _merge_into(dst, src, n, op):
    def body(i, _):
        s = pl.ds(i * LANES, LANES)
        dst[s] = op(dst[s], src[s])
        return 0

    lax.fori_loop(0, n // LANES, body, 0)


def _scatter_max(ref, idx16, val16, mask=None):
    pend = val16 > plsc.load_gather(ref, [idx16])
    if mask is not None:
        pend = pend & mask

    def cond(p):
        return jnp.any(p)

    def step(p):
        plsc.store_scatter(ref, [idx16], val16, mask=p)
        newp = val16 > plsc.load_gather(ref, [idx16])
        if mask is not None:
            newp = newp & mask
        return newp

    lax.while_loop(cond, step, pend)


def _reduce_tiles(local, sh, out_hbm, n_pad, cid, sid, red, tmp, rps, op):
    pltpu.sync_copy(local, sh.at[sid])
    plsc.subcore_barrier()
    sl = pl.ds(sid * rps, rps)
    pltpu.sync_copy(sh.at[0, sl], red)
    for t in range(1, NS):
        pltpu.sync_copy(sh.at[t, sl], tmp)
        _merge_into(red, tmp, rps, op)
    pltpu.sync_copy(red, out_hbm.at[pl.ds(cid * n_pad + sid * rps, rps)])
    plsc.subcore_barrier()


def _edge_scatter(src, row, col, n_pad):
    n, d = src.shape
    e = row.shape[0]
    per_w = e // NW
    blk_sz = 96
    nfull = per_w // blk_sz
    tail = per_w % blk_sz
    rps = n_pad // NS

    @functools.partial(
        pl.kernel,
        out_type=jax.ShapeDtypeStruct((NC, n_pad, d), jnp.float32),
        mesh=_MESH,
        scratch_types=[
            pltpu.VMEM((per_w,), jnp.int32),
            pltpu.VMEM((per_w,), jnp.int32),
            pltpu.VMEM((blk_sz,), jnp.int32),
            pltpu.VMEM((blk_sz,), jnp.int32),
            pltpu.VMEM((blk_sz, d), jnp.float32),
            pltpu.VMEM((blk_sz, d), jnp.float32),
            pltpu.VMEM((max(tail, 8),), jnp.int32),
            pltpu.VMEM((max(tail, 8),), jnp.int32),
            pltpu.VMEM_SHARED((n_pad, d), jnp.float32),
            pltpu.SemaphoreType.DMA,
            pltpu.SemaphoreType.DMA,
        ],
        compiler_params=_SC_PARAMS,
    )
    def k(src_hbm, row_hbm, col_hbm, out_hbm, ridx_all, cidx_all, rbuf, cbuf,
          buf_a, buf_b, rt, ct, acc, sem_a, sem_b):
        cid = lax.axis_index("c")
        sid = lax.axis_index("s")
        wid = sid * NC + cid

        def zrow(i, _):
            for j in range(d // LANES):
                buf_a[i, pl.ds(j * LANES, LANES)] = jnp.zeros((LANES,),
                                                              jnp.float32)
            return 0

        lax.fori_loop(0, blk_sz, zrow, 0)
        off = 0
        while off < rps:
            step = min(blk_sz, rps - off)
            pltpu.sync_copy(buf_a.at[pl.ds(0, step)],
                            acc.at[pl.ds(sid * rps + off, step)])
            off += step
        plsc.subcore_barrier()

        wbase = wid * per_w
        pltpu.sync_copy(row_hbm.at[pl.ds(wbase, per_w)], ridx_all)
        pltpu.sync_copy(col_hbm.at[pl.ds(wbase, per_w)], cidx_all)

        def blk(j, _):
            for g in range(blk_sz // LANES):
                s = pl.ds(g * LANES, LANES)
                rbuf[s] = ridx_all[pl.ds(j * blk_sz + g * LANES, LANES)]
                cbuf[s] = cidx_all[pl.ds(j * blk_sz + g * LANES, LANES)]
            da = pltpu.async_copy(src_hbm.at[rbuf], buf_a, sem_a)
            db = pltpu.async_copy(src_hbm.at[cbuf], buf_b, sem_b)
            da.wait()
            pltpu.sync_copy(buf_a, acc.at[cbuf], add=True)
            db.wait()
            pltpu.sync_copy(buf_b, acc.at[rbuf], add=True)
            return 0

        lax.fori_loop(0, nfull, blk, 0)
        if tail:
            base = nfull * blk_sz
            for g in range(tail // LANES):
                s = pl.ds(g * LANES, LANES)
                rt[s] = ridx_all[pl.ds(base + g * LANES, LANES)]
                ct[s] = cidx_all[pl.ds(base + g * LANES, LANES)]
            ta = buf_a.at[pl.ds(0, tail)]
            tb = buf_b.at[pl.ds(0, tail)]
            da = pltpu.async_copy(src_hbm.at[rt], ta, sem_a)
            db = pltpu.async_copy(src_hbm.at[ct], tb, sem_b)
            da.wait()
            pltpu.sync_copy(ta, acc.at[ct], add=True)
            db.wait()
            pltpu.sync_copy(tb, acc.at[rt], add=True)
        plsc.subcore_barrier()
        sl = pl.ds(sid * rps, rps)
        pltpu.sync_copy(acc.at[sl], out_hbm.at[cid, sl])

    return k(src, row, col)


def _row_scatter(src_pad, cluster, n_pad):
    n, d = src_pad.shape
    nblk = n // 128
    rps = n_pad // NS

    @functools.partial(
        pl.kernel,
        out_type=jax.ShapeDtypeStruct((NC, n_pad, d), jnp.float32),
        mesh=_MESH,
        scratch_types=[
            pltpu.VMEM((128,), jnp.int32),
            pltpu.VMEM((128, d), jnp.float32),
            pltpu.VMEM_SHARED((n_pad, d), jnp.float32),
        ],
        compiler_params=_SC_PARAMS,
    )
    def k(src_hbm, cl_hbm, out_hbm, cidx, buf, acc):
        cid = lax.axis_index("c")
        sid = lax.axis_index("s")
        wid = sid * NC + cid

        def zrow(i, _):
            for j in range(d // LANES):
                buf[i, pl.ds(j * LANES, LANES)] = jnp.zeros((LANES,), jnp.float32)
            return 0

        lax.fori_loop(0, 128, zrow, 0)
        for kk in range(rps // 128):
            pltpu.sync_copy(buf, acc.at[pl.ds(sid * rps + kk * 128, 128)])
        plsc.subcore_barrier()

        nb = (nblk - wid + NW - 1) // NW

        def blk(j, _):
            b = wid + j * NW
            pltpu.sync_copy(cl_hbm.at[pl.ds(b * 128, 128)], cidx)
            pltpu.sync_copy(src_hbm.at[pl.ds(b * 128, 128)], buf)
            pltpu.sync_copy(buf, acc.at[cidx], add=True)
            return 0

        lax.fori_loop(0, nb, blk, 0)
        plsc.subcore_barrier()
        sl = pl.ds(sid * rps, rps)
        pltpu.sync_copy(acc.at[sl], out_hbm.at[cid, sl])

    return k(src_pad, cluster)


def _cluster_pass1(row, col, ns_pad, n_pad):
    e = row.shape[0]
    per_w = e // NW
    nfull = per_w // 128
    tail = per_w % 128
    rps = n_pad // NS

    @functools.partial(
        pl.kernel,
        out_type=(
            jax.ShapeDtypeStruct((NC * n_pad,), jnp.float32),
            jax.ShapeDtypeStruct((NC * n_pad,), jnp.float32),
        ),
        mesh=_MESH,
        scratch_types=[
            pltpu.VMEM((n_pad,), jnp.float32),
            pltpu.VMEM((n_pad,), jnp.float32),
            pltpu.VMEM((n_pad,), jnp.float32),
            pltpu.VMEM((per_w,), jnp.int32),
            pltpu.VMEM((per_w,), jnp.int32),
            pltpu.VMEM((rps,), jnp.float32),
            pltpu.VMEM((rps,), jnp.float32),
            pltpu.VMEM_SHARED((NS, n_pad), jnp.float32),
        ],
        compiler_params=_SC_PARAMS,
    )
    def k(row_hbm, col_hbm, ns_hbm, best_out, deg_out, ns_l, best_l, deg_l,
          ridx_all, cidx_all, red, tmp, sh):
        cid = lax.axis_index("c")
        sid = lax.axis_index("s")
        wid = sid * NC + cid
        ones16 = jnp.ones((LANES,), jnp.float32)

        wbase = wid * per_w
        pltpu.sync_copy(row_hbm.at[pl.ds(wbase, per_w)], ridx_all)
        pltpu.sync_copy(col_hbm.at[pl.ds(wbase, per_w)], cidx_all)
        pltpu.sync_copy(ns_hbm, ns_l)
        _fill(best_l, n_pad, -1.0, jnp.float32)
        _fill(deg_l, n_pad, 0.0, jnp.float32)

        def grp(g, _):
            s = pl.ds(g * LANES, LANES)
            r16 = ridx_all[s]
            c16 = cidx_all[s]
            es = plsc.load_gather(ns_l, [r16]) * plsc.load_gather(ns_l, [c16])
            _scatter_max(best_l, r16, es)
            _scatter_max(best_l, c16, es)
            plsc.addupdate_scatter(deg_l, [r16], ones16)
            plsc.addupdate_scatter(deg_l, [c16], ones16)
            return 0

        lax.fori_loop(0, per_w // LANES, grp, 0)

        _reduce_tiles(best_l, sh, best_out, n_pad, cid, sid, red, tmp, rps,
                      jnp.maximum)
        _reduce_tiles(deg_l, sh, deg_out, n_pad, cid, sid, red, tmp, rps,
                      jnp.add)

    return k(row, col, ns_pad)


def _cluster_pass2(row, col, ns_pad, best_part, n_pad):
    e = row.shape[0]
    per_w = e // NW
    nfull = per_w // 128
    tail = per_w % 128
    rps = n_pad // NS

    @functools.partial(
        pl.kernel,
        out_type=jax.ShapeDtypeStruct((NC * n_pad,), jnp.int32),
        mesh=_MESH,
        scratch_types=[
            pltpu.VMEM((n_pad,), jnp.float32),
            pltpu.VMEM((n_pad,), jnp.float32),
            pltpu.VMEM((n_pad,), jnp.float32),
            pltpu.VMEM((n_pad,), jnp.int32),
            pltpu.VMEM((per_w,), jnp.int32),
            pltpu.VMEM((per_w,), jnp.int32),
            pltpu.VMEM((rps,), jnp.int32),
            pltpu.VMEM((rps,), jnp.int32),
            pltpu.VMEM_SHARED((NS, n_pad), jnp.int32),
        ],
        compiler_params=_SC_PARAMS,
    )
    def k(row_hbm, col_hbm, ns_hbm, best_hbm, par_out, ns_l, best_l, btmp,
          parent_l, ridx_all, cidx_all, red, tmp, sh):
        cid = lax.axis_index("c")
        sid = lax.axis_index("s")
        wid = sid * NC + cid

        wbase = wid * per_w
        pltpu.sync_copy(row_hbm.at[pl.ds(wbase, per_w)], ridx_all)
        pltpu.sync_copy(col_hbm.at[pl.ds(wbase, per_w)], cidx_all)
        pltpu.sync_copy(ns_hbm, ns_l)
        pltpu.sync_copy(best_hbm.at[pl.ds(0, n_pad)], best_l)
        pltpu.sync_copy(best_hbm.at[pl.ds(n_pad, n_pad)], btmp)
        _merge_into(best_l, btmp, n_pad, jnp.maximum)
        _fill(parent_l, n_pad, -1, jnp.int32)

        def grp(g, _):
            s = pl.ds(g * LANES, LANES)
            r16 = ridx_all[s]
            c16 = cidx_all[s]
            es = plsc.load_gather(ns_l, [r16]) * plsc.load_gather(ns_l, [c16])
            isb_r = es >= plsc.load_gather(best_l, [r16])
            _scatter_max(parent_l, r16, c16, mask=isb_r)
            isb_c = es >= plsc.load_gather(best_l, [c16])
            _scatter_max(parent_l, c16, r16, mask=isb_c)
            return 0

        lax.fori_loop(0, per_w // LANES, grp, 0)

        _reduce_tiles(parent_l, sh, par_out, n_pad, cid, sid, red, tmp, rps,
                      jnp.maximum)

    return k(row, col, ns_pad, best_part)


def _cluster_pass3(parent_part, deg_part, ns_pad, n_pad):
    rps = n_pad // NS
    npw = n_pad // NW

    @functools.partial(
        pl.kernel,
        out_type=(
            jax.ShapeDtypeStruct((n_pad,), jnp.int32),
            jax.ShapeDtypeStruct((NC * n_pad,), jnp.float32),
            jax.ShapeDtypeStruct((NC * n_pad,), jnp.float32),
            jax.ShapeDtypeStruct((NC * n_pad,), jnp.float32),
        ),
        mesh=_MESH,
        scratch_types=[
            pltpu.VMEM((n_pad,), jnp.int32),
            pltpu.VMEM((n_pad,), jnp.int32),
            pltpu.VMEM((n_pad,), jnp.int32),
            pltpu.VMEM((n_pad,), jnp.float32),
            pltpu.VMEM((n_pad,), jnp.float32),
            pltpu.VMEM((n_pad,), jnp.float32),
            pltpu.VMEM((npw,), jnp.int32),
            pltpu.VMEM((npw,), jnp.float32),
            pltpu.VMEM((npw,), jnp.float32),
            pltpu.VMEM((npw,), jnp.float32),
            pltpu.VMEM((rps,), jnp.float32),
            pltpu.VMEM((rps,), jnp.float32),
            pltpu.VMEM_SHARED((NS, n_pad), jnp.float32),
        ],
        compiler_params=_SC_PARAMS,
    )
    def k(par_hbm, deg_hbm, ns_hbm, cl_out, cs_out, lk_out, sp_out,
          parent_l, itmp, cluster1_l, csize_l, link_l, spool_l,
          clbuf, dbuf, dtmp, nbuf, red, tmp, sh):
        cid = lax.axis_index("c")
        sid = lax.axis_index("s")
        wid = sid * NC + cid
        ones16 = jnp.ones((LANES,), jnp.float32)

        pltpu.sync_copy(par_hbm.at[pl.ds(0, n_pad)], parent_l)
        pltpu.sync_copy(par_hbm.at[pl.ds(n_pad, n_pad)], itmp)
        _merge_into(parent_l, itmp, n_pad, jnp.maximum)

        iota16 = lax.iota(jnp.int32, LANES)

        def c1(i, _):
            s = pl.ds(i * LANES, LANES)
            idxv = iota16 + i * LANES
            p = parent_l[s]
            p = jnp.where(p < 0, idxv, p)
            cluster1_l[s] = jnp.minimum(idxv, p)
            return 0

        lax.fori_loop(0, n_pad // LANES, c1, 0)

        _fill(csize_l, n_pad, 0.0, jnp.float32)
        _fill(link_l, n_pad, 0.0, jnp.float32)
        _fill(spool_l, n_pad, -1.0, jnp.float32)

        base = wid * npw
        pltpu.sync_copy(deg_hbm.at[pl.ds(base, npw)], dbuf)
        pltpu.sync_copy(deg_hbm.at[pl.ds(n_pad + base, npw)], dtmp)
        _merge_into(dbuf, dtmp, npw, jnp.add)
        pltpu.sync_copy(ns_hbm.at[pl.ds(base, npw)], nbuf)

        for g in range(npw // LANES):
            s = pl.ds(g * LANES, LANES)
            cl1v = cluster1_l[pl.ds(base + g * LANES, LANES)]
            clp = plsc.load_gather(cluster1_l, [cl1v])
            cl = jnp.minimum(cl1v, clp)
            clbuf[s] = cl
            plsc.addupdate_scatter(csize_l, [cl], ones16)
            plsc.addupdate_scatter(link_l, [cl], dbuf[s])
            _scatter_max(spool_l, cl, nbuf[s])

        pltpu.sync_copy(clbuf, cl_out.at[pl.ds(base, npw)])

        _reduce_tiles(csize_l, sh, cs_out, n_pad, cid, sid, red, tmp, rps,
                      jnp.add)
        _reduce_tiles(link_l, sh, lk_out, n_pad, cid, sid, red, tmp, rps,
                      jnp.add)
        _reduce_tiles(spool_l, sh, sp_out, n_pad, cid, sid, red, tmp, rps,
                      jnp.maximum)

    return k(parent_part, deg_part, ns_pad)


def _mlp2(h, w0, b0, w1, b1):
    h = jnp.maximum(jnp.dot(h, w0, preferred_element_type=jnp.float32) + b0, 0.0)
    return jnp.dot(h, w1, preferred_element_type=jnp.float32) + b1


def _full_spec(shape):
    nd = len(shape)
    return pl.BlockSpec(shape, lambda i: (0,) * nd)


def _tc_in_mlp(x, w0, b0, w1, b1, br):
    n, d = x.shape

    def body(x_r, w0_r, b0_r, w1_r, b1_r, o_r):
        o_r[...] = _mlp2(x_r[...], w0_r[...], b0_r[...], w1_r[...], b1_r[...])

    return pl.pallas_call(
        body,
        grid=(n // br,),
        in_specs=[
            pl.BlockSpec((br, d), lambda i: (i, 0)),
            _full_spec(w0.shape), _full_spec(b0.shape),
            _full_spec(w1.shape), _full_spec(b1.shape),
        ],
        out_specs=pl.BlockSpec((br, d), lambda i: (i, 0)),
        out_shape=jax.ShapeDtypeStruct((n, d), jnp.float32),
    )(x, w0, b0, w1, b1)


def _tc_gnn_layer(g, agg_part, w, b, br):
    n, d = g.shape

    def body(g_r, a0_r, a1_r, w_r, b_r, o_r):
        s = g_r[...] + a0_r[0] + a1_r[0]
        o_r[...] = jnp.maximum(
            jnp.dot(s, w_r[...], preferred_element_type=jnp.float32) + b_r[...],
            0.0)

    return pl.pallas_call(
        body,
        grid=(n // br,),
        in_specs=[
            pl.BlockSpec((br, d), lambda i: (i, 0)),
            pl.BlockSpec((1, br, d), lambda i: (0, i, 0)),
            pl.BlockSpec((1, br, d), lambda i: (1, i, 0)),
            _full_spec(w.shape), _full_spec(b.shape),
        ],
        out_specs=pl.BlockSpec((br, d), lambda i: (i, 0)),
        out_shape=jax.ShapeDtypeStruct((n, d), jnp.float32),
    )(g, agg_part, agg_part, w, b)


def _tc_gnn2_score_pre(g, agg_part, w, b, sw, sb, pw0, pb0, pw1, pb1, br):
    n, d = g.shape

    def body(g_r, a0_r, a1_r, w_r, b_r, sw_r, sb_r, pw0_r, pb0_r, pw1_r,
             pb1_r, ns_r, pre_r):
        s = g_r[...] + a0_r[0] + a1_r[0]
        g2 = jnp.maximum(
            jnp.dot(s, w_r[...], preferred_element_type=jnp.float32) + b_r[...],
            0.0)
        z = jnp.dot(g2, sw_r[...], preferred_element_type=jnp.float32) + sb_r[...]
        ns_r[...] = jax.nn.sigmoid(z)
        pre_r[...] = _mlp2(g2, pw0_r[...], pb0_r[...], pw1_r[...], pb1_r[...])

    return pl.pallas_call(
        body,
        grid=(n // br,),
        in_specs=[
            pl.BlockSpec((br, d), lambda i: (i, 0)),
            pl.BlockSpec((1, br, d), lambda i: (0, i, 0)),
            pl.BlockSpec((1, br, d), lambda i: (1, i, 0)),
            _full_spec(w.shape), _full_spec(b.shape),
            _full_spec(sw.shape), _full_spec(sb.shape),
            _full_spec(pw0.shape), _full_spec(pb0.shape),
            _full_spec(pw1.shape), _full_spec(pb1.shape),
        ],
        out_specs=(
            pl.BlockSpec((br, 1), lambda i: (i, 0)),
            pl.BlockSpec((br, d), lambda i: (i, 0)),
        ),
        out_shape=(
            jax.ShapeDtypeStruct((n, 1), jnp.float32),
            jax.ShapeDtypeStruct((n, d), jnp.float32),
        ),
    )(g, agg_part, agg_part, w, b, sw, sb, pw0, pb0, pw1, pb1)


def _tc_final(pooled_part, pw0, pb0, pw1, pb1, sp, lk, cs, h, ow0, ob0, ow1,
              ob1, br):
    n, d = h.shape

    def body(q0_r, q1_r, pw0_r, pb0_r, pw1_r, pb1_r, sp0_r, sp1_r, lk0_r,
             lk1_r, cs0_r, cs1_r, h_r, ow0_r, ob0_r, ow1_r, ob1_r, o_r):
        pooled = _mlp2(q0_r[0] + q1_r[0], pw0_r[...], pb0_r[...], pw1_r[...],
                       pb1_r[...])
        csv = cs0_r[0] + cs1_r[0]
        spool = jnp.maximum(sp0_r[0], sp1_r[0])
        spool = jnp.where(csv > 0.0, spool, 0.0)
        lkv = lk0_r[0] + lk1_r[0]
        pooled = pooled * spool * jnp.log1p(lkv)
        ho = jnp.where(csv == 1.0, h_r[...], pooled)
        o_r[...] = _mlp2(ho, ow0_r[...], ob0_r[...], ow1_r[...], ob1_r[...])

    def part2d(i_sel):
        return pl.BlockSpec((1, br, d), lambda i, s=i_sel: (s, i, 0))

    def part1d(i_sel):
        return pl.BlockSpec((1, br, 1), lambda i, s=i_sel: (s, i, 0))

    return pl.pallas_call(
        body,
        grid=(n // br,),
        in_specs=[
            part2d(0), part2d(1),
            _full_spec(pw0.shape), _full_spec(pb0.shape),
            _full_spec(pw1.shape), _full_spec(pb1.shape),
            part1d(0), part1d(1), part1d(0), part1d(1), part1d(0), part1d(1),
            pl.BlockSpec((br, d), lambda i: (i, 0)),
            _full_spec(ow0.shape), _full_spec(ob0.shape),
            _full_spec(ow1.shape), _full_spec(ob1.shape),
        ],
        out_specs=pl.BlockSpec((br, d), lambda i: (i, 0)),
        out_shape=jax.ShapeDtypeStruct((n, d), jnp.float32),
    )(pooled_part, pooled_part, pw0, pb0, pw1, pb1, sp, sp, lk, lk, cs, cs,
      h, ow0, ob0, ow1, ob1)


def kernel(x, edge_index, batch, params):
    n, d = x.shape
    n_pad = -(-n // 2048) * 2048
    row = edge_index[0]
    col = edge_index[1]
    p = params
    br = 2000 if n % 2000 == 0 else 400

    def b2d(b):
        return b.reshape(1, d)

    in_w, in_b = p["in_W"], p["in_b"]
    gnn_w, gnn_b = p["gnn_W"], p["gnn_b"]
    pre_w, pre_b = p["pre_W"], p["pre_b"]
    post_w, post_b = p["post_W"], p["post_b"]
    out_w, out_b = p["out_W"], p["out_b"]
    sw = p["score_w"].reshape(d, 1)
    sb = p["score_b"].reshape(1, 1)

    h = _tc_in_mlp(x, in_w[0], b2d(in_b[0]), in_w[1], b2d(in_b[1]), br)
    agg1 = _edge_scatter(h, row, col, n_pad)
    g1 = _tc_gnn_layer(h, agg1, gnn_w[0], b2d(gnn_b[0]), br)
    agg2 = _edge_scatter(g1, row, col, n_pad)
    ns, pre = _tc_gnn2_score_pre(
        g1, agg2, gnn_w[1], b2d(gnn_b[1]), sw, sb,
        pre_w[0], b2d(pre_b[0]), pre_w[1], b2d(pre_b[1]), br)

    ns_pad = jnp.pad(ns.reshape(-1), (0, n_pad - n))
    best_part, deg_part = _cluster_pass1(row, col, ns_pad, n_pad)
    parent_part = _cluster_pass2(row, col, ns_pad, best_part, n_pad)
    cluster_pad, cs_part, lk_part, sp_part = _cluster_pass3(
        parent_part, deg_part, ns_pad, n_pad)

    pre_pad = jnp.pad(pre, ((0, n_pad - n), (0, 0)))
    pooled_part = _row_scatter(pre_pad, cluster_pad, n_pad)

    out = _tc_final(
        pooled_part, post_w[0], b2d(post_b[0]), post_w[1], b2d(post_b[1]),
        sp_part.reshape(NC, n_pad, 1), lk_part.reshape(NC, n_pad, 1),
        cs_part.reshape(NC, n_pad, 1), h,
        out_w[0], b2d(out_b[0]), out_w[1], b2d(out_b[1]), br)
    return out, cluster_pad[:n]
